# baseline (speedup 1.0000x reference)
import jax
import jax.numpy as jnp
from jax import lax
from jax.experimental import pallas as pl
from jax.experimental.pallas import tpu as pltpu
from jax.experimental.pallas import tpu_sc as plsc

_NC = 2
_NS = 16
_NW = _NC * _NS
_C = 16
_NBX = 6
_NBP = 3
_LANES = 16


def _make_sc_add(B, S, D):
    s_per_w = S // _NW
    n_chunks = s_per_w // _C
    total = n_chunks * B
    assert S % _NW == 0 and s_per_w % _C == 0 and D % _LANES == 0
    mesh = plsc.VectorSubcoreMesh(
        core_axis_name="c", subcore_axis_name="s",
        num_cores=_NC, num_subcores=_NS,
    )

    def body(x_hbm, pos_hbm, out_hbm, Xs, Ps, sem_x, sem_p, sem_o):
        wid = lax.axis_index("s") * _NC + lax.axis_index("c")
        sbase = wid * s_per_w

        def start_p(i):
            return pltpu.async_copy(
                pos_hbm.at[pl.ds(sbase + i * _C, _C)], Ps[i % _NBP], sem_p)

        def row0(step):
            i, b = divmod(step, B)
            return b * S + sbase + i * _C

        def start_x(step):
            return pltpu.async_copy(
                x_hbm.at[pl.ds(row0(step), _C)], Xs[step % _NBX], sem_x)

        cp_p = {0: start_p(0), 1: start_p(1)}
        cp_x = {s: start_x(s) for s in range(_NBX - 2)}
        cp_o = {}
        step = 0
        for i in range(n_chunks):
            if i + 2 < n_chunks:
                cp_p[i + 2] = start_p(i + 2)
            for b in range(B):
                if step >= 2:
                    cp_o[step - 2].wait()
                if step + _NBX - 2 < total:
                    cp_x[step + _NBX - 2] = start_x(step + _NBX - 2)
                if b == 0:
                    cp_p[i].wait()
                cp_x[step].wait()
                Xc, Pc = Xs[step % _NBX], Ps[i % _NBP]

                @plsc.parallel_loop(0, _C * D, _LANES, unroll=8)
                def addbody(j):
                    r = j // D
                    c = j - r * D
                    plsc.addupdate(
                        Xc.at[r, pl.ds(c, _LANES)], Pc[r, pl.ds(c, _LANES)])

                cp_o[step] = pltpu.async_copy(
                    Xc, out_hbm.at[pl.ds(row0(step), _C)], sem_o)
                step += 1
        cp_o[total - 2].wait()
        cp_o[total - 1].wait()

    def body_wrap(x_hbm, pos_hbm, out_hbm, *scratch):
        Xs = list(scratch[:_NBX])
        Ps = list(scratch[_NBX:_NBX + _NBP])
        sem_x, sem_p, sem_o = scratch[_NBX + _NBP:]
        body(x_hbm, pos_hbm, out_hbm, Xs, Ps, sem_x, sem_p, sem_o)

    return pl.kernel(
        body_wrap,
        out_type=jax.ShapeDtypeStruct((B * S, D), jnp.float32),
        mesh=mesh,
        scratch_types=(
            [pltpu.VMEM((_C, D), jnp.float32)] * (_NBX + _NBP)
            + [pltpu.SemaphoreType.DMA] * 3
        ),
    )


def kernel(x, pos_table):
    B, S, D = x.shape
    out = _make_sc_add(B, S, D)(x.reshape(B * S, D), pos_table)
    return out.reshape(B, S, D)
